# baseline (speedup 1.0000x reference)
import functools

import jax
import jax.numpy as jnp
from jax import lax
from jax.experimental import pallas as pl
from jax.experimental.pallas import tpu as pltpu
from jax.experimental.pallas import tpu_sc as plsc

N = 10000
E = 320000
D = 128
NC, NS = 2, 16
NW = NC * NS
CH = 64
EPAD = 2560 * 128
NCK = EPAD // CH // NW
BR = 400
NRC = N // BR
NPAD = 10240
PK = NPAD // 8
TREP = 256

_mesh = plsc.VectorSubcoreMesh(core_axis_name="c", subcore_axis_name="s")


def _worker_id():
    return lax.axis_index("s") * NC + lax.axis_index("c")


def _copy_rows(src_at, dst_at, base, total, piece):
    off = 0
    while off < total:
        n = min(piece, total - off)
        pltpu.sync_copy(src_at(base + off, n), dst_at(base + off, n))
        off += n


def _over_row_chunks(s, fn):
    fn(s * BR)

    @pl.when(s < NRC - NS)
    def _():
        fn((s + NS) * BR)


NCH = 2
NCKC = NCK // NCH


def _make_scatter(acc_rows, out_rows, table_rows=None):
    zps = acc_rows // NS
    contiguous_out = out_rows % (NS * 8) == 0

    @functools.partial(
        pl.kernel,
        out_type=jax.ShapeDtypeStruct((NC, out_rows, D), jnp.float32),
        mesh=_mesh,
        scratch_types=(
            [pltpu.VMEM((CH,), jnp.int32)] * (8 * NCH)
            + [pltpu.VMEM((CH, D), jnp.float32)] * (2 * NCH)
            + [pltpu.VMEM_SHARED((acc_rows, D), jnp.float32)]
            + ([pltpu.VMEM_SHARED((table_rows, D), jnp.float32)]
               if table_rows else [])
            + [pltpu.SemaphoreType.DMA] * (12 * NCH)
        ),
    )
    def _scatter(g_hbm, src_hbm, dst_hbm, out_hbm, *scr):
        idxrefs = scr[:8 * NCH]
        rowrefs = scr[8 * NCH:8 * NCH + 2 * NCH]
        acc = scr[8 * NCH + 2 * NCH]
        sems = scr[8 * NCH + 2 * NCH + (2 if table_rows else 1):]
        c = lax.axis_index("c")
        s = lax.axis_index("s")
        wid = _worker_id()

        if table_rows:
            tbl = scr[8 * NCH + 2 * NCH + 1]
            trs = table_rows // NS
            pltpu.sync_copy(g_hbm.at[pl.ds(s * trs, trs)],
                            tbl.at[pl.ds(s * trs, trs)])
            gsrc = tbl
        else:
            gsrc = g_hbm

        zero16 = jnp.zeros((16,), jnp.float32)
        rows0 = rowrefs[0]

        def zrow(i, _):
            for j in range(D // 16):
                rows0[i, pl.ds(j * 16, 16)] = zero16
            return _

        lax.fori_loop(0, CH, zrow, 0)
        _copy_rows(lambda b, n: rows0.at[pl.ds(0, n)],
                   lambda b, n: acc.at[pl.ds(b, n)], s * zps, zps, CH)
        plsc.subcore_barrier()

        def make_chain(t):
            IS = idxrefs[t * 8:t * 8 + 4]
            ID = idxrefs[t * 8 + 4:t * 8 + 8]
            RW = rowrefs[t * 2:t * 2 + 2]
            SIS = sems[t * 12:t * 12 + 4]
            SID = sems[t * 12 + 4:t * 12 + 8]
            SG = sems[t * 12 + 8:t * 12 + 10]
            SS = sems[t * 12 + 10:t * 12 + 12]
            cbase = wid * NCK + t * NCKC

            def off(k):
                return (cbase + k) * CH

            def idx_start(k, r):
                pltpu.async_copy(src_hbm.at[pl.ds(off(k), CH)], IS[r], SIS[r])
                pltpu.async_copy(dst_hbm.at[pl.ds(off(k), CH)], ID[r], SID[r])

            def idx_s_wait(r):
                pltpu.make_async_copy(
                    src_hbm.at[pl.ds(0, CH)], IS[r], SIS[r]).wait()

            def idx_d_wait(r):
                pltpu.make_async_copy(
                    dst_hbm.at[pl.ds(0, CH)], ID[r], SID[r]).wait()

            def gather_start(r, p):
                pltpu.async_copy(gsrc.at[IS[r]], RW[p], SG[p])

            def gather_wait(r, p):
                pltpu.make_async_copy(gsrc.at[IS[r]], RW[p], SG[p]).wait()

            def scat_start(r, p):
                pltpu.async_copy(RW[p], acc.at[ID[r]], SS[p], add=True)

            def scat_wait(r, p):
                pltpu.make_async_copy(RW[p], acc.at[ID[r]], SS[p]).wait()

            def body(k, b, first, last, look):
                p, q, r = b & 1, 1 - (b & 1), b & 3
                gather_wait(r, p)
                idx_d_wait(r)
                scat_start(r, p)
                if not first:
                    scat_wait((b - 1) & 3, q)
                if not last:
                    idx_s_wait((b + 1) & 3)
                    gather_start((b + 1) & 3, q)
                if look:
                    idx_start(k + 3, (b + 3) & 3)

            def prime():
                for j in range(3):
                    idx_start(j, j)
                idx_s_wait(0)
                gather_start(0, 0)

            def drain():
                scat_wait((NCKC - 1) & 3, (NCKC - 1) & 1)

            return body, prime, drain

        chains = [make_chain(t) for t in range(NCH)]
        for body, prime, _ in chains:
            prime()
        for j in range(4):
            for body, _, _ in chains:
                body(j, j, j == 0, False, True)

        def loop(i, carry):
            k = 4 * i
            for b in range(4):
                for body, _, _ in chains:
                    body(k + b, b, False, False, True)
            return carry

        lax.fori_loop(1, NCKC // 4 - 1, loop, 0)

        for j in range(NCKC - 4, NCKC):
            for body, _, _ in chains:
                body(j, j & 3, False, j >= NCKC - 1, j + 3 < NCKC)
        for _, _, drain in chains:
            drain()

        plsc.subcore_barrier()
        if contiguous_out:
            ops = out_rows // NS
            _copy_rows(lambda b, n: acc.at[pl.ds(b, n)],
                       lambda b, n: out_hbm.at[c, pl.ds(b, n)],
                       s * ops, ops, CH)
        else:
            _over_row_chunks(s, lambda base: _copy_rows(
                lambda b, n: acc.at[pl.ds(b, n)],
                lambda b, n: out_hbm.at[c, pl.ds(b, n)], base, BR, CH))

    return _scatter


_agg_sc = _make_scatter(NPAD, N)
_deg_sc = _make_scatter(PK, PK, table_rows=TREP * 8)


def _dis(deg_ref):
    i = pl.program_id(0)
    dd = deg_ref[pl.ds(i * BR, BR), :]
    return lax.rsqrt((dd[:, 0:1] + dd[:, 1:2]) * (1.0 / 16.0) + 1.0)


def _mm_tc(x_ref, w_ref, o_ref):
    o_ref[...] = jnp.dot(x_ref[...], w_ref[...],
                         preferred_element_type=jnp.float32)


def _scale_tc(deg_ref, p_ref, o_ref):
    o_ref[...] = p_ref[...] * _dis(deg_ref)


def _mid_tc(deg_ref, s_ref, g_ref, b_ref, w_ref, o_ref):
    dis = _dis(deg_ref)
    h = dis * (s_ref[0] + s_ref[1] + g_ref[...]) + b_ref[...]
    h = jnp.where(h >= 0.0, h, 0.01 * h)
    o_ref[...] = jnp.dot(h, w_ref[...],
                         preferred_element_type=jnp.float32) * dis


def _last_tc(deg_ref, s_ref, g_ref, b_ref, w_ref, bfc_ref, o_ref):
    dis = _dis(deg_ref)
    h = dis * (s_ref[0] + s_ref[1] + g_ref[...]) + b_ref[...]
    h = jnp.where(h >= 0.0, h, 0.01 * h)
    o_ref[...] = jnp.dot(h, w_ref[...],
                         preferred_element_type=jnp.float32) + bfc_ref[...]


_GRID = (N // BR,)
_deg_spec = pl.BlockSpec((N, 2), lambda i: (0, 0))
_row_spec = pl.BlockSpec((BR, D), lambda i: (i, 0))
_s_spec = pl.BlockSpec((NC, BR, D), lambda i: (0, i, 0))
_w_spec = pl.BlockSpec((D, D), lambda i: (0, 0))
_b_spec = pl.BlockSpec((1, D), lambda i: (0, 0))
_out_shape = jax.ShapeDtypeStruct((N, D), jnp.float32)

_mm_call = pl.pallas_call(
    _mm_tc, grid=_GRID,
    in_specs=[_row_spec, _w_spec],
    out_specs=_row_spec, out_shape=_out_shape)

_scale_call = pl.pallas_call(
    _scale_tc, grid=_GRID,
    in_specs=[_deg_spec, _row_spec],
    out_specs=_row_spec, out_shape=_out_shape)

_mid_call = pl.pallas_call(
    _mid_tc, grid=_GRID,
    in_specs=[_deg_spec, _s_spec, _row_spec, _b_spec, _w_spec],
    out_specs=_row_spec, out_shape=_out_shape)

_last_call = pl.pallas_call(
    _last_tc, grid=_GRID,
    in_specs=[_deg_spec, _s_spec, _row_spec, _b_spec, _w_spec, _b_spec],
    out_specs=_row_spec, out_shape=_out_shape)


def kernel(x, edge_index, W1, b1, W2, b2, Wfc, bfc):
    ei = edge_index.astype(jnp.int32)
    pad = EPAD - E
    r = jnp.arange(pad, dtype=jnp.int32)
    src = jnp.concatenate([ei[0], r * 41 % N])
    dst = jnp.concatenate([ei[1], N + r % (NPAD - N)])
    hi = jnp.right_shift(dst, 3)
    lo = jnp.bitwise_and(dst, TREP * 8 - 1)
    onehot = jnp.tile(jnp.repeat(jnp.eye(8, dtype=jnp.float32), 16, axis=1),
                      (TREP, 1))
    p1 = _mm_call(x, W1)
    degp = _deg_sc(onehot, lo, hi)
    deg = degp.reshape(NC, NPAD, 16).sum(2)[:, :N]
    degT = jnp.transpose(deg)
    g1 = _scale_call(degT, p1)
    s1 = _agg_sc(g1, src, dst)
    g2 = _mid_call(degT, s1, g1, b1.reshape(1, D), W2)
    s2 = _agg_sc(g2, src, dst)
    out = _last_call(degT, s2, g2, b2.reshape(1, D), Wfc, bfc.reshape(1, D))
    return out


# speedup vs baseline: 1.0030x; 1.0030x over previous
import functools

import jax
import jax.numpy as jnp
from jax import lax
from jax.experimental import pallas as pl
from jax.experimental.pallas import tpu as pltpu
from jax.experimental.pallas import tpu_sc as plsc

N = 10000
E = 320000
D = 128
NC, NS = 2, 16
NW = NC * NS
CH = 64
EPAD = 2560 * 128
NCK = EPAD // CH // NW
BR = 400
NRC = N // BR
NPAD = 10240
PK = NPAD // 8
TREP = 256

_mesh = plsc.VectorSubcoreMesh(core_axis_name="c", subcore_axis_name="s")


def _worker_id():
    return lax.axis_index("s") * NC + lax.axis_index("c")


def _copy_rows(src_at, dst_at, base, total, piece):
    off = 0
    while off < total:
        n = min(piece, total - off)
        pltpu.sync_copy(src_at(base + off, n), dst_at(base + off, n))
        off += n


def _over_row_chunks(s, fn):
    fn(s * BR)

    @pl.when(s < NRC - NS)
    def _():
        fn((s + NS) * BR)


NCH = 2
NCKC = NCK // NCH


def _make_scatter(acc_rows, out_rows, table_rows=None):
    zps = acc_rows // NS
    contiguous_out = out_rows % (NS * 8) == 0

    @functools.partial(
        pl.kernel,
        out_type=jax.ShapeDtypeStruct((NC, out_rows, D), jnp.float32),
        mesh=_mesh,
        scratch_types=(
            [pltpu.VMEM((CH,), jnp.int32)] * (8 * NCH)
            + [pltpu.VMEM((CH, D), jnp.float32)] * (2 * NCH)
            + [pltpu.VMEM_SHARED((acc_rows, D), jnp.float32)]
            + ([pltpu.VMEM_SHARED((table_rows, D), jnp.float32)]
               if table_rows else [])
            + [pltpu.SemaphoreType.DMA] * (12 * NCH)
        ),
    )
    def _scatter(g_hbm, src_hbm, dst_hbm, out_hbm, *scr):
        idxrefs = scr[:8 * NCH]
        rowrefs = scr[8 * NCH:8 * NCH + 2 * NCH]
        acc = scr[8 * NCH + 2 * NCH]
        sems = scr[8 * NCH + 2 * NCH + (2 if table_rows else 1):]
        c = lax.axis_index("c")
        s = lax.axis_index("s")
        wid = _worker_id()

        if table_rows:
            tbl = scr[8 * NCH + 2 * NCH + 1]
            trs = table_rows // NS
            pltpu.sync_copy(g_hbm.at[pl.ds(s * trs, trs)],
                            tbl.at[pl.ds(s * trs, trs)])
            gsrc = tbl
        else:
            gsrc = g_hbm

        zero16 = jnp.zeros((16,), jnp.float32)
        rows0 = rowrefs[0]

        def zrow(i, _):
            for j in range(D // 16):
                rows0[i, pl.ds(j * 16, 16)] = zero16
            return _

        lax.fori_loop(0, CH, zrow, 0)
        _copy_rows(lambda b, n: rows0.at[pl.ds(0, n)],
                   lambda b, n: acc.at[pl.ds(b, n)], s * zps, zps, CH)
        plsc.subcore_barrier()

        def make_chain(t):
            IS = idxrefs[t * 8:t * 8 + 4]
            ID = idxrefs[t * 8 + 4:t * 8 + 8]
            RW = rowrefs[t * 2:t * 2 + 2]
            SIS = sems[t * 12:t * 12 + 4]
            SID = sems[t * 12 + 4:t * 12 + 8]
            SG = sems[t * 12 + 8:t * 12 + 10]
            SS = sems[t * 12 + 10:t * 12 + 12]
            cbase = wid * NCK + t * NCKC

            def off(k):
                return (cbase + k) * CH

            def idx_start(k, r):
                pltpu.async_copy(src_hbm.at[pl.ds(off(k), CH)], IS[r], SIS[r])
                pltpu.async_copy(dst_hbm.at[pl.ds(off(k), CH)], ID[r], SID[r])

            def idx_s_wait(r):
                pltpu.make_async_copy(
                    src_hbm.at[pl.ds(0, CH)], IS[r], SIS[r]).wait()

            def idx_d_wait(r):
                pltpu.make_async_copy(
                    dst_hbm.at[pl.ds(0, CH)], ID[r], SID[r]).wait()

            def gather_start(r, p):
                pltpu.async_copy(gsrc.at[IS[r]], RW[p], SG[p])

            def gather_wait(r, p):
                pltpu.make_async_copy(gsrc.at[IS[r]], RW[p], SG[p]).wait()

            def scat_start(r, p):
                pltpu.async_copy(RW[p], acc.at[ID[r]], SS[p], add=True)

            def scat_wait(r, p):
                pltpu.make_async_copy(RW[p], acc.at[ID[r]], SS[p]).wait()

            def body(k, b, first, last, look):
                p, q, r = b & 1, 1 - (b & 1), b & 3
                gather_wait(r, p)
                idx_d_wait(r)
                scat_start(r, p)
                if not first:
                    scat_wait((b - 1) & 3, q)
                if not last:
                    idx_s_wait((b + 1) & 3)
                    gather_start((b + 1) & 3, q)
                if look:
                    idx_start(k + 3, (b + 3) & 3)

            def prime():
                for j in range(3):
                    idx_start(j, j)
                idx_s_wait(0)
                gather_start(0, 0)

            def drain():
                scat_wait((NCKC - 1) & 3, (NCKC - 1) & 1)

            return body, prime, drain

        chains = [make_chain(t) for t in range(NCH)]
        for body, prime, _ in chains:
            prime()
        for j in range(4):
            for body, _, _ in chains:
                body(j, j, j == 0, False, True)

        def loop(i, carry):
            k = 4 * i
            for b in range(4):
                for body, _, _ in chains:
                    body(k + b, b, False, False, True)
            return carry

        lax.fori_loop(1, NCKC // 4 - 1, loop, 0)

        for j in range(NCKC - 4, NCKC):
            for body, _, _ in chains:
                body(j, j & 3, False, j >= NCKC - 1, j + 3 < NCKC)
        for _, _, drain in chains:
            drain()

        plsc.subcore_barrier()
        if contiguous_out:
            ops = out_rows // NS
            _copy_rows(lambda b, n: acc.at[pl.ds(b, n)],
                       lambda b, n: out_hbm.at[c, pl.ds(b, n)],
                       s * ops, ops, CH)
        else:
            _over_row_chunks(s, lambda base: _copy_rows(
                lambda b, n: acc.at[pl.ds(b, n)],
                lambda b, n: out_hbm.at[c, pl.ds(b, n)], base, BR, CH))

    return _scatter


_agg_sc = _make_scatter(NPAD, N)
_deg_sc = _make_scatter(PK, PK, table_rows=TREP * 8)


def _dis(deg_ref):
    return lax.rsqrt(
        (deg_ref[:, 0:1] + deg_ref[:, 1:2]) * (1.0 / 16.0) + 1.0)


def _mm_tc(x_ref, w_ref, o_ref):
    o_ref[...] = jnp.dot(x_ref[...], w_ref[...],
                         preferred_element_type=jnp.float32)


def _scale_tc(deg_ref, p_ref, o_ref):
    o_ref[...] = p_ref[...] * _dis(deg_ref)


def _mid_tc(deg_ref, s_ref, g_ref, b_ref, w_ref, o_ref):
    dis = _dis(deg_ref)
    h = dis * (s_ref[0] + s_ref[1] + g_ref[...]) + b_ref[...]
    h = jnp.where(h >= 0.0, h, 0.01 * h)
    o_ref[...] = jnp.dot(h, w_ref[...],
                         preferred_element_type=jnp.float32) * dis


def _last_tc(deg_ref, s_ref, g_ref, b_ref, w_ref, bfc_ref, o_ref):
    dis = _dis(deg_ref)
    h = dis * (s_ref[0] + s_ref[1] + g_ref[...]) + b_ref[...]
    h = jnp.where(h >= 0.0, h, 0.01 * h)
    o_ref[...] = jnp.dot(h, w_ref[...],
                         preferred_element_type=jnp.float32) + bfc_ref[...]


_GRID = (N // BR,)
_deg_spec = pl.BlockSpec((BR, 2), lambda i: (i, 0))
_row_spec = pl.BlockSpec((BR, D), lambda i: (i, 0))
_s_spec = pl.BlockSpec((NC, BR, D), lambda i: (0, i, 0))
_w_spec = pl.BlockSpec((D, D), lambda i: (0, 0))
_b_spec = pl.BlockSpec((1, D), lambda i: (0, 0))
_out_shape = jax.ShapeDtypeStruct((N, D), jnp.float32)

_mm_call = pl.pallas_call(
    _mm_tc, grid=_GRID,
    in_specs=[_row_spec, _w_spec],
    out_specs=_row_spec, out_shape=_out_shape)

_scale_call = pl.pallas_call(
    _scale_tc, grid=_GRID,
    in_specs=[_deg_spec, _row_spec],
    out_specs=_row_spec, out_shape=_out_shape)

_mid_call = pl.pallas_call(
    _mid_tc, grid=_GRID,
    in_specs=[_deg_spec, _s_spec, _row_spec, _b_spec, _w_spec],
    out_specs=_row_spec, out_shape=_out_shape)

_last_call = pl.pallas_call(
    _last_tc, grid=_GRID,
    in_specs=[_deg_spec, _s_spec, _row_spec, _b_spec, _w_spec, _b_spec],
    out_specs=_row_spec, out_shape=_out_shape)


def kernel(x, edge_index, W1, b1, W2, b2, Wfc, bfc):
    ei = edge_index.astype(jnp.int32)
    pad = EPAD - E
    r = jnp.arange(pad, dtype=jnp.int32)
    src = jnp.concatenate([ei[0], r * 41 % N])
    dst = jnp.concatenate([ei[1], N + r % (NPAD - N)])
    hi = jnp.right_shift(dst, 3)
    lo = jnp.bitwise_and(dst, TREP * 8 - 1)
    onehot = jnp.tile(jnp.repeat(jnp.eye(8, dtype=jnp.float32), 16, axis=1),
                      (TREP, 1))
    p1 = _mm_call(x, W1)
    degp = _deg_sc(onehot, lo, hi)
    deg = degp.reshape(NC, NPAD, 16).sum(2)[:, :N]
    degT = jnp.transpose(deg)
    g1 = _scale_call(degT, p1)
    s1 = _agg_sc(g1, src, dst)
    g2 = _mid_call(degT, s1, g1, b1.reshape(1, D), W2)
    s2 = _agg_sc(g2, src, dst)
    out = _last_call(degT, s2, g2, b2.reshape(1, D), Wfc, bfc.reshape(1, D))
    return out


# speedup vs baseline: 1.0253x; 1.0223x over previous
import functools

import jax
import jax.numpy as jnp
from jax import lax
from jax.experimental import pallas as pl
from jax.experimental.pallas import tpu as pltpu
from jax.experimental.pallas import tpu_sc as plsc

N = 10000
E = 320000
D = 128
NC, NS = 2, 16
NW = NC * NS
CH = 64
EPAD = 2560 * 128
NCK = EPAD // CH // NW
BR = 400
NRC = N // BR
NPAD = 10240
PK = NPAD // 8
TREP = 256

_mesh = plsc.VectorSubcoreMesh(core_axis_name="c", subcore_axis_name="s")


def _worker_id():
    return lax.axis_index("s") * NC + lax.axis_index("c")


def _copy_rows(src_at, dst_at, base, total, piece):
    off = 0
    while off < total:
        n = min(piece, total - off)
        pltpu.sync_copy(src_at(base + off, n), dst_at(base + off, n))
        off += n


def _over_row_chunks(s, fn):
    fn(s * BR)

    @pl.when(s < NRC - NS)
    def _():
        fn((s + NS) * BR)


NCH = 2
NCKC = NCK // NCH


def _make_scatter(acc_rows, out_rows, table_rows=None):
    zps = acc_rows // NS
    contiguous_out = out_rows % (NS * 8) == 0

    @functools.partial(
        pl.kernel,
        out_type=jax.ShapeDtypeStruct((NC, out_rows, D), jnp.float32),
        mesh=_mesh,
        scratch_types=(
            [pltpu.VMEM((CH,), jnp.int32)] * (8 * NCH)
            + [pltpu.VMEM((CH, D), jnp.float32)] * (2 * NCH)
            + [pltpu.VMEM_SHARED((acc_rows, D), jnp.float32)]
            + ([pltpu.VMEM_SHARED((table_rows, D), jnp.float32)]
               if table_rows else [])
            + [pltpu.SemaphoreType.DMA] * (12 * NCH)
        ),
    )
    def _scatter(g_hbm, src_hbm, dst_hbm, out_hbm, *scr):
        idxrefs = scr[:8 * NCH]
        rowrefs = scr[8 * NCH:8 * NCH + 2 * NCH]
        acc = scr[8 * NCH + 2 * NCH]
        sems = scr[8 * NCH + 2 * NCH + (2 if table_rows else 1):]
        c = lax.axis_index("c")
        s = lax.axis_index("s")
        wid = _worker_id()

        if table_rows:
            tbl = scr[8 * NCH + 2 * NCH + 1]
            trs = table_rows // NS
            pltpu.sync_copy(g_hbm.at[pl.ds(s * trs, trs)],
                            tbl.at[pl.ds(s * trs, trs)])
            gsrc = tbl
        else:
            gsrc = g_hbm

        zero16 = jnp.zeros((16,), jnp.float32)
        rows0 = rowrefs[0]

        def zrow(i, _):
            for j in range(D // 16):
                rows0[i, pl.ds(j * 16, 16)] = zero16
            return _

        lax.fori_loop(0, CH, zrow, 0)
        _copy_rows(lambda b, n: rows0.at[pl.ds(0, n)],
                   lambda b, n: acc.at[pl.ds(b, n)], s * zps, zps, CH)
        plsc.subcore_barrier()

        def make_chain(t):
            IS = idxrefs[t * 8:t * 8 + 4]
            ID = idxrefs[t * 8 + 4:t * 8 + 8]
            RW = rowrefs[t * 2:t * 2 + 2]
            SIS = sems[t * 12:t * 12 + 4]
            SID = sems[t * 12 + 4:t * 12 + 8]
            SG = sems[t * 12 + 8:t * 12 + 10]
            SS = sems[t * 12 + 10:t * 12 + 12]
            cbase = wid * NCK + t * NCKC

            def off(k):
                return (cbase + k) * CH

            def idx_start(k, r):
                pltpu.async_copy(src_hbm.at[pl.ds(off(k), CH)], IS[r], SIS[r])
                pltpu.async_copy(dst_hbm.at[pl.ds(off(k), CH)], ID[r], SID[r])

            def idx_s_wait(r):
                pltpu.make_async_copy(
                    src_hbm.at[pl.ds(0, CH)], IS[r], SIS[r]).wait()

            def idx_d_wait(r):
                pltpu.make_async_copy(
                    dst_hbm.at[pl.ds(0, CH)], ID[r], SID[r]).wait()

            def gather_start(r, p):
                pltpu.async_copy(gsrc.at[IS[r]], RW[p], SG[p])

            def gather_wait(r, p):
                pltpu.make_async_copy(gsrc.at[IS[r]], RW[p], SG[p]).wait()

            def scat_start(r, p):
                pltpu.async_copy(RW[p], acc.at[ID[r]], SS[p], add=True)

            def scat_wait(r, p):
                pltpu.make_async_copy(RW[p], acc.at[ID[r]], SS[p]).wait()

            def body(k, b, first, last, look):
                p, q, r = b & 1, 1 - (b & 1), b & 3
                gather_wait(r, p)
                idx_d_wait(r)
                scat_start(r, p)
                if not first:
                    scat_wait((b - 1) & 3, q)
                if not last:
                    idx_s_wait((b + 1) & 3)
                    gather_start((b + 1) & 3, q)
                if look:
                    idx_start(k + 3, (b + 3) & 3)

            def prime():
                for j in range(3):
                    idx_start(j, j)
                idx_s_wait(0)
                gather_start(0, 0)

            def drain():
                scat_wait((NCKC - 1) & 3, (NCKC - 1) & 1)

            return body, prime, drain

        chains = [make_chain(t) for t in range(NCH)]
        for body, prime, _ in chains:
            prime()
        for j in range(4):
            for body, _, _ in chains:
                body(j, j, j == 0, False, True)

        def loop(i, carry):
            k = 4 * i
            for b in range(4):
                for body, _, _ in chains:
                    body(k + b, b, False, False, True)
            return carry

        lax.fori_loop(1, NCKC // 4 - 1, loop, 0)

        for j in range(NCKC - 4, NCKC):
            for body, _, _ in chains:
                body(j, j & 3, False, j >= NCKC - 1, j + 3 < NCKC)
        for _, _, drain in chains:
            drain()

        plsc.subcore_barrier()
        if contiguous_out:
            ops = out_rows // NS
            _copy_rows(lambda b, n: acc.at[pl.ds(b, n)],
                       lambda b, n: out_hbm.at[c, pl.ds(b, n)],
                       s * ops, ops, CH)
        else:
            _over_row_chunks(s, lambda base: _copy_rows(
                lambda b, n: acc.at[pl.ds(b, n)],
                lambda b, n: out_hbm.at[c, pl.ds(b, n)], base, BR, CH))

    return _scatter


_agg_sc = _make_scatter(NPAD, N)
_deg_sc = _make_scatter(PK, PK, table_rows=TREP * 8)


def _dis(deg_ref):
    return lax.rsqrt(deg_ref[:, 0:1] + deg_ref[:, 1:2] + 1.0)


def _mm_tc(x_ref, w_ref, o_ref):
    o_ref[...] = jnp.dot(x_ref[...], w_ref[...],
                         preferred_element_type=jnp.float32)


def _scale_tc(deg_ref, p_ref, o_ref):
    o_ref[...] = p_ref[...] * _dis(deg_ref)


def _mid_tc(deg_ref, s_ref, g_ref, b_ref, w_ref, o_ref):
    dis = _dis(deg_ref)
    h = dis * (s_ref[0] + s_ref[1] + g_ref[...]) + b_ref[...]
    h = jnp.where(h >= 0.0, h, 0.01 * h)
    o_ref[...] = jnp.dot(h, w_ref[...],
                         preferred_element_type=jnp.float32) * dis


def _last_tc(deg_ref, s_ref, g_ref, b_ref, w_ref, bfc_ref, o_ref):
    dis = _dis(deg_ref)
    h = dis * (s_ref[0] + s_ref[1] + g_ref[...]) + b_ref[...]
    h = jnp.where(h >= 0.0, h, 0.01 * h)
    o_ref[...] = jnp.dot(h, w_ref[...],
                         preferred_element_type=jnp.float32) + bfc_ref[...]


_GRID = (N // BR,)
_deg_spec = pl.BlockSpec((BR, 2), lambda i: (i, 0))
_row_spec = pl.BlockSpec((BR, D), lambda i: (i, 0))
_s_spec = pl.BlockSpec((NC, BR, D), lambda i: (0, i, 0))
_w_spec = pl.BlockSpec((D, D), lambda i: (0, 0))
_b_spec = pl.BlockSpec((1, D), lambda i: (0, 0))
_out_shape = jax.ShapeDtypeStruct((N, D), jnp.float32)

_mm_call = pl.pallas_call(
    _mm_tc, grid=_GRID,
    in_specs=[_row_spec, _w_spec],
    out_specs=_row_spec, out_shape=_out_shape)

_scale_call = pl.pallas_call(
    _scale_tc, grid=_GRID,
    in_specs=[_deg_spec, _row_spec],
    out_specs=_row_spec, out_shape=_out_shape)

_mid_call = pl.pallas_call(
    _mid_tc, grid=_GRID,
    in_specs=[_deg_spec, _s_spec, _row_spec, _b_spec, _w_spec],
    out_specs=_row_spec, out_shape=_out_shape)

_last_call = pl.pallas_call(
    _last_tc, grid=_GRID,
    in_specs=[_deg_spec, _s_spec, _row_spec, _b_spec, _w_spec, _b_spec],
    out_specs=_row_spec, out_shape=_out_shape)


def kernel(x, edge_index, W1, b1, W2, b2, Wfc, bfc):
    ei = edge_index.astype(jnp.int32)
    pad = EPAD - E
    r = jnp.arange(pad, dtype=jnp.int32)
    src = jnp.concatenate([ei[0], r * 41 % N])
    dst = jnp.concatenate([ei[1], N + r % (NPAD - N)])
    hi = jnp.right_shift(dst, 3)
    lo = jnp.bitwise_and(dst, TREP * 8 - 1)
    onehot = jnp.tile(jnp.repeat(jnp.eye(8, dtype=jnp.float32), 16, axis=1),
                      (TREP, 1))
    p1 = _mm_call(x, W1)
    degp = _deg_sc(onehot, lo, hi)
    deg = degp.reshape(NC, PK, 8, 16)[:, :, :, 0].reshape(NC, NPAD)[:, :N]
    degT = jnp.transpose(deg)
    g1 = _scale_call(degT, p1)
    s1 = _agg_sc(g1, src, dst)
    g2 = _mid_call(degT, s1, g1, b1.reshape(1, D), W2)
    s2 = _agg_sc(g2, src, dst)
    out = _last_call(degT, s2, g2, b2.reshape(1, D), Wfc, bfc.reshape(1, D))
    return out


# speedup vs baseline: 1.0957x; 1.0686x over previous
import functools

import jax
import jax.numpy as jnp
from jax import lax
from jax.experimental import pallas as pl
from jax.experimental.pallas import tpu as pltpu
from jax.experimental.pallas import tpu_sc as plsc

N = 10000
E = 320000
D = 128
NC, NS = 2, 16
NW = NC * NS
CH = 64
EPAD = 2560 * 128
NCK = EPAD // CH // NW
BR = 400
NRC = N // BR
NPAD = 10240
PK = NPAD // 8
TREP = 256

_mesh = plsc.VectorSubcoreMesh(core_axis_name="c", subcore_axis_name="s")


def _worker_id():
    return lax.axis_index("s") * NC + lax.axis_index("c")


def _copy_rows(src_at, dst_at, base, total, piece):
    off = 0
    while off < total:
        n = min(piece, total - off)
        pltpu.sync_copy(src_at(base + off, n), dst_at(base + off, n))
        off += n


def _over_row_chunks(s, fn):
    fn(s * BR)

    @pl.when(s < NRC - NS)
    def _():
        fn((s + NS) * BR)


NCH = 2
NCKC = NCK // NCH


def _make_scatter(acc_rows, out_rows, table_rows=None):
    zps = acc_rows // NS
    contiguous_out = out_rows % (NS * 8) == 0

    @functools.partial(
        pl.kernel,
        out_type=jax.ShapeDtypeStruct((NC, out_rows, D), jnp.float32),
        mesh=_mesh,
        scratch_types=(
            [pltpu.VMEM((CH,), jnp.int32)] * (8 * NCH)
            + [pltpu.VMEM((CH, D), jnp.float32)] * (2 * NCH)
            + [pltpu.VMEM_SHARED((acc_rows, D), jnp.float32)]
            + ([pltpu.VMEM_SHARED((table_rows, D), jnp.float32)]
               if table_rows else [])
            + [pltpu.SemaphoreType.DMA] * (12 * NCH)
        ),
    )
    def _scatter(g_hbm, src_hbm, dst_hbm, out_hbm, *scr):
        idxrefs = scr[:8 * NCH]
        rowrefs = scr[8 * NCH:8 * NCH + 2 * NCH]
        acc = scr[8 * NCH + 2 * NCH]
        sems = scr[8 * NCH + 2 * NCH + (2 if table_rows else 1):]
        c = lax.axis_index("c")
        s = lax.axis_index("s")
        wid = _worker_id()

        if table_rows:
            tbl = scr[8 * NCH + 2 * NCH + 1]
            trs = table_rows // NS
            pltpu.sync_copy(g_hbm.at[pl.ds(s * trs, trs)],
                            tbl.at[pl.ds(s * trs, trs)])
            gsrc = tbl
        else:
            gsrc = g_hbm

        zero16 = jnp.zeros((16,), jnp.float32)
        rows0 = rowrefs[0]

        def zrow(i, _):
            for j in range(D // 16):
                rows0[i, pl.ds(j * 16, 16)] = zero16
            return _

        lax.fori_loop(0, CH, zrow, 0)
        _copy_rows(lambda b, n: rows0.at[pl.ds(0, n)],
                   lambda b, n: acc.at[pl.ds(b, n)], s * zps, zps, CH)
        plsc.subcore_barrier()

        def make_chain(t):
            IS = idxrefs[t * 8:t * 8 + 4]
            ID = idxrefs[t * 8 + 4:t * 8 + 8]
            RW = rowrefs[t * 2:t * 2 + 2]
            SIS = sems[t * 12:t * 12 + 4]
            SID = sems[t * 12 + 4:t * 12 + 8]
            SG = sems[t * 12 + 8:t * 12 + 10]
            SS = sems[t * 12 + 10:t * 12 + 12]
            cbase = wid * NCK + t * NCKC

            def off(k):
                return (cbase + k) * CH

            def idx_start(k, r):
                pltpu.async_copy(src_hbm.at[pl.ds(off(k), CH)], IS[r], SIS[r])
                pltpu.async_copy(dst_hbm.at[pl.ds(off(k), CH)], ID[r], SID[r])

            def idx_s_wait(r):
                pltpu.make_async_copy(
                    src_hbm.at[pl.ds(0, CH)], IS[r], SIS[r]).wait()

            def idx_d_wait(r):
                pltpu.make_async_copy(
                    dst_hbm.at[pl.ds(0, CH)], ID[r], SID[r]).wait()

            def gather_start(r, p):
                pltpu.async_copy(gsrc.at[IS[r]], RW[p], SG[p])

            def gather_wait(r, p):
                pltpu.make_async_copy(gsrc.at[IS[r]], RW[p], SG[p]).wait()

            def scat_start(r, p):
                pltpu.async_copy(RW[p], acc.at[ID[r]], SS[p], add=True)

            def scat_wait(r, p):
                pltpu.make_async_copy(RW[p], acc.at[ID[r]], SS[p]).wait()

            def body(k, b, first, last, look):
                p, q, r = b & 1, 1 - (b & 1), b & 3
                gather_wait(r, p)
                idx_d_wait(r)
                scat_start(r, p)
                if not first:
                    scat_wait((b - 1) & 3, q)
                if not last:
                    idx_s_wait((b + 1) & 3)
                    gather_start((b + 1) & 3, q)
                if look:
                    idx_start(k + 3, (b + 3) & 3)

            def prime():
                for j in range(3):
                    idx_start(j, j)
                idx_s_wait(0)
                gather_start(0, 0)

            def drain():
                scat_wait((NCKC - 1) & 3, (NCKC - 1) & 1)

            return body, prime, drain

        chains = [make_chain(t) for t in range(NCH)]
        for body, prime, _ in chains:
            prime()
        for j in range(4):
            for body, _, _ in chains:
                body(j, j, j == 0, False, True)

        def loop(i, carry):
            k = 4 * i
            for b in range(4):
                for body, _, _ in chains:
                    body(k + b, b, False, False, True)
            return carry

        lax.fori_loop(1, NCKC // 4 - 1, loop, 0)

        for j in range(NCKC - 4, NCKC):
            for body, _, _ in chains:
                body(j, j & 3, False, j >= NCKC - 1, j + 3 < NCKC)
        for _, _, drain in chains:
            drain()

        plsc.subcore_barrier()
        if contiguous_out:
            ops = out_rows // NS
            _copy_rows(lambda b, n: acc.at[pl.ds(b, n)],
                       lambda b, n: out_hbm.at[c, pl.ds(b, n)],
                       s * ops, ops, CH)
        else:
            _over_row_chunks(s, lambda base: _copy_rows(
                lambda b, n: acc.at[pl.ds(b, n)],
                lambda b, n: out_hbm.at[c, pl.ds(b, n)], base, BR, CH))

    return _scatter


_agg_sc = _make_scatter(NPAD, N)
_deg_sc = _make_scatter(PK, PK, table_rows=TREP * 8)


def _dis(deg_ref):
    return lax.rsqrt(deg_ref[:, 0:1] + deg_ref[:, 1:2] + 1.0)


def _mm_tc(x_ref, w_ref, o_ref):
    o_ref[...] = jnp.dot(x_ref[...], w_ref[...],
                         preferred_element_type=jnp.float32)


def _scale_tc(deg_ref, p_ref, o_ref):
    o_ref[...] = p_ref[...] * _dis(deg_ref)


def _mid_tc(deg_ref, s_ref, g_ref, b_ref, w_ref, o_ref):
    dis = _dis(deg_ref)
    h = dis * (s_ref[0] + s_ref[1] + g_ref[...]) + b_ref[...]
    h = jnp.where(h >= 0.0, h, 0.01 * h)
    o_ref[...] = jnp.dot(h, w_ref[...],
                         preferred_element_type=jnp.float32) * dis


def _last_tc(deg_ref, s_ref, g_ref, b_ref, w_ref, bfc_ref, o_ref):
    dis = _dis(deg_ref)
    h = dis * (s_ref[0] + s_ref[1] + g_ref[...]) + b_ref[...]
    h = jnp.where(h >= 0.0, h, 0.01 * h)
    o_ref[...] = jnp.dot(h, w_ref[...],
                         preferred_element_type=jnp.float32) + bfc_ref[...]


TBR = 2000
_GRID = (N // TBR,)
_deg_spec = pl.BlockSpec((TBR, 2), lambda i: (i, 0))
_row_spec = pl.BlockSpec((TBR, D), lambda i: (i, 0))
_s_spec = pl.BlockSpec((NC, TBR, D), lambda i: (0, i, 0))
_w_spec = pl.BlockSpec((D, D), lambda i: (0, 0))
_b_spec = pl.BlockSpec((1, D), lambda i: (0, 0))
_out_shape = jax.ShapeDtypeStruct((N, D), jnp.float32)

_mm_call = pl.pallas_call(
    _mm_tc, grid=_GRID,
    in_specs=[_row_spec, _w_spec],
    out_specs=_row_spec, out_shape=_out_shape)

_scale_call = pl.pallas_call(
    _scale_tc, grid=_GRID,
    in_specs=[_deg_spec, _row_spec],
    out_specs=_row_spec, out_shape=_out_shape)

_mid_call = pl.pallas_call(
    _mid_tc, grid=_GRID,
    in_specs=[_deg_spec, _s_spec, _row_spec, _b_spec, _w_spec],
    out_specs=_row_spec, out_shape=_out_shape)

_last_call = pl.pallas_call(
    _last_tc, grid=_GRID,
    in_specs=[_deg_spec, _s_spec, _row_spec, _b_spec, _w_spec, _b_spec],
    out_specs=_row_spec, out_shape=_out_shape)


def kernel(x, edge_index, W1, b1, W2, b2, Wfc, bfc):
    ei = edge_index.astype(jnp.int32)
    pad = EPAD - E
    r = jnp.arange(pad, dtype=jnp.int32)
    src = jnp.concatenate([ei[0], r * 41 % N])
    dst = jnp.concatenate([ei[1], N + r % (NPAD - N)])
    hi = jnp.right_shift(dst, 3)
    lo = jnp.bitwise_and(dst, TREP * 8 - 1)
    onehot = jnp.tile(jnp.repeat(jnp.eye(8, dtype=jnp.float32), 16, axis=1),
                      (TREP, 1))
    p1 = _mm_call(x, W1)
    degp = _deg_sc(onehot, lo, hi)
    deg = degp.reshape(NC, PK, 8, 16)[:, :, :, 0].reshape(NC, NPAD)[:, :N]
    degT = jnp.transpose(deg)
    g1 = _scale_call(degT, p1)
    s1 = _agg_sc(g1, src, dst)
    g2 = _mid_call(degT, s1, g1, b1.reshape(1, D), W2)
    s2 = _agg_sc(g2, src, dst)
    out = _last_call(degT, s2, g2, b2.reshape(1, D), Wfc, bfc.reshape(1, D))
    return out


# speedup vs baseline: 1.1076x; 1.0108x over previous
import functools

import jax
import jax.numpy as jnp
from jax import lax
from jax.experimental import pallas as pl
from jax.experimental.pallas import tpu as pltpu
from jax.experimental.pallas import tpu_sc as plsc

N = 10000
E = 320000
D = 128
NC, NS = 2, 16
NW = NC * NS
CH = 64
EPAD = 2560 * 128
NCK = EPAD // CH // NW
BR = 400
NRC = N // BR
NPAD = 10240
PK = NPAD // 8
TREP = 256

_mesh = plsc.VectorSubcoreMesh(core_axis_name="c", subcore_axis_name="s")


def _worker_id():
    return lax.axis_index("s") * NC + lax.axis_index("c")


def _copy_rows(src_at, dst_at, base, total, piece):
    off = 0
    while off < total:
        n = min(piece, total - off)
        pltpu.sync_copy(src_at(base + off, n), dst_at(base + off, n))
        off += n


def _over_row_chunks(s, fn):
    fn(s * BR)

    @pl.when(s < NRC - NS)
    def _():
        fn((s + NS) * BR)


NCH = 2
NCKC = NCK // NCH


def _make_scatter(acc_rows, out_rows, table_rows=None):
    zps = acc_rows // NS
    contiguous_out = out_rows % (NS * 8) == 0

    @functools.partial(
        pl.kernel,
        out_type=jax.ShapeDtypeStruct((NC, out_rows, D), jnp.float32),
        mesh=_mesh,
        scratch_types=(
            [pltpu.VMEM((CH,), jnp.int32)] * (8 * NCH)
            + [pltpu.VMEM((CH, D), jnp.float32)] * (2 * NCH)
            + [pltpu.VMEM_SHARED((acc_rows, D), jnp.float32)]
            + ([pltpu.VMEM_SHARED((table_rows, D), jnp.float32)]
               if table_rows else [])
            + [pltpu.SemaphoreType.DMA] * (12 * NCH)
        ),
    )
    def _scatter(g_hbm, src_hbm, dst_hbm, out_hbm, *scr):
        idxrefs = scr[:8 * NCH]
        rowrefs = scr[8 * NCH:8 * NCH + 2 * NCH]
        acc = scr[8 * NCH + 2 * NCH]
        sems = scr[8 * NCH + 2 * NCH + (2 if table_rows else 1):]
        c = lax.axis_index("c")
        s = lax.axis_index("s")
        wid = _worker_id()

        if table_rows:
            tbl = scr[8 * NCH + 2 * NCH + 1]
            trs = table_rows // NS
            pltpu.sync_copy(g_hbm.at[pl.ds(s * trs, trs)],
                            tbl.at[pl.ds(s * trs, trs)])
            gsrc = tbl
        else:
            gsrc = g_hbm

        zero16 = jnp.zeros((16,), jnp.float32)
        rows0 = rowrefs[0]

        def zrow(i, _):
            for j in range(D // 16):
                rows0[i, pl.ds(j * 16, 16)] = zero16
            return _

        lax.fori_loop(0, CH, zrow, 0)
        _copy_rows(lambda b, n: rows0.at[pl.ds(0, n)],
                   lambda b, n: acc.at[pl.ds(b, n)], s * zps, zps, CH)
        plsc.subcore_barrier()

        def make_chain(t):
            IS = idxrefs[t * 8:t * 8 + 4]
            ID = idxrefs[t * 8 + 4:t * 8 + 8]
            RW = rowrefs[t * 2:t * 2 + 2]
            SIS = sems[t * 12:t * 12 + 4]
            SID = sems[t * 12 + 4:t * 12 + 8]
            SG = sems[t * 12 + 8:t * 12 + 10]
            SS = sems[t * 12 + 10:t * 12 + 12]
            cbase = wid * NCK + t * NCKC

            def off(k):
                return (cbase + k) * CH

            def idx_start(k, r):
                pltpu.async_copy(src_hbm.at[pl.ds(off(k), CH)], IS[r], SIS[r])
                pltpu.async_copy(dst_hbm.at[pl.ds(off(k), CH)], ID[r], SID[r])

            def idx_s_wait(r):
                pltpu.make_async_copy(
                    src_hbm.at[pl.ds(0, CH)], IS[r], SIS[r]).wait()

            def idx_d_wait(r):
                pltpu.make_async_copy(
                    dst_hbm.at[pl.ds(0, CH)], ID[r], SID[r]).wait()

            def gather_start(r, p):
                pltpu.async_copy(gsrc.at[IS[r]], RW[p], SG[p])

            def gather_wait(r, p):
                pltpu.make_async_copy(gsrc.at[IS[r]], RW[p], SG[p]).wait()

            def scat_start(r, p):
                pltpu.async_copy(RW[p], acc.at[ID[r]], SS[p], add=True)

            def scat_wait(r, p):
                pltpu.make_async_copy(RW[p], acc.at[ID[r]], SS[p]).wait()

            def body(k, b, first, last, look):
                p, q, r = b & 1, 1 - (b & 1), b & 3
                gather_wait(r, p)
                idx_d_wait(r)
                scat_start(r, p)
                if not first:
                    scat_wait((b - 1) & 3, q)
                if not last:
                    idx_s_wait((b + 1) & 3)
                    gather_start((b + 1) & 3, q)
                if look:
                    idx_start(k + 3, (b + 3) & 3)

            def prime():
                for j in range(3):
                    idx_start(j, j)
                idx_s_wait(0)
                gather_start(0, 0)

            def drain():
                scat_wait((NCKC - 1) & 3, (NCKC - 1) & 1)

            return body, prime, drain

        chains = [make_chain(t) for t in range(NCH)]
        for body, prime, _ in chains:
            prime()
        for j in range(4):
            for body, _, _ in chains:
                body(j, j, j == 0, False, True)

        def loop(i, carry):
            k = 4 * i
            for b in range(4):
                for body, _, _ in chains:
                    body(k + b, b, False, False, True)
            return carry

        lax.fori_loop(1, NCKC // 4 - 1, loop, 0)

        for j in range(NCKC - 4, NCKC):
            for body, _, _ in chains:
                body(j, j & 3, False, j >= NCKC - 1, j + 3 < NCKC)
        for _, _, drain in chains:
            drain()

        plsc.subcore_barrier()
        if contiguous_out:
            ops = out_rows // NS
            _copy_rows(lambda b, n: acc.at[pl.ds(b, n)],
                       lambda b, n: out_hbm.at[c, pl.ds(b, n)],
                       s * ops, ops, CH)
        else:
            _over_row_chunks(s, lambda base: _copy_rows(
                lambda b, n: acc.at[pl.ds(b, n)],
                lambda b, n: out_hbm.at[c, pl.ds(b, n)], base, BR, CH))

    return _scatter


_agg_sc = _make_scatter(NPAD, N)
_deg_sc = _make_scatter(PK, PK, table_rows=TREP * 8)


def _dis(deg_ref):
    return lax.rsqrt(deg_ref[:, 0:1] + deg_ref[:, 1:2] + 1.0)


def _mm_tc(x_ref, w_ref, o_ref):
    o_ref[...] = jnp.dot(x_ref[...], w_ref[...],
                         preferred_element_type=jnp.float32)


def _scale_tc(deg_ref, p_ref, o_ref):
    o_ref[...] = p_ref[...] * _dis(deg_ref)


def _mid_tc(deg_ref, s_ref, g_ref, b_ref, w_ref, o_ref):
    dis = _dis(deg_ref)
    h = dis * (s_ref[0] + s_ref[1] + g_ref[...]) + b_ref[...]
    h = jnp.where(h >= 0.0, h, 0.01 * h)
    o_ref[...] = jnp.dot(h, w_ref[...],
                         preferred_element_type=jnp.float32) * dis


def _last_tc(deg_ref, s_ref, g_ref, b_ref, w_ref, bfc_ref, o_ref):
    dis = _dis(deg_ref)
    h = dis * (s_ref[0] + s_ref[1] + g_ref[...]) + b_ref[...]
    h = jnp.where(h >= 0.0, h, 0.01 * h)
    o_ref[...] = jnp.dot(h, w_ref[...],
                         preferred_element_type=jnp.float32) + bfc_ref[...]


TBR = 5000
_GRID = (N // TBR,)
_deg_spec = pl.BlockSpec((TBR, 2), lambda i: (i, 0))
_row_spec = pl.BlockSpec((TBR, D), lambda i: (i, 0))
_s_spec = pl.BlockSpec((NC, TBR, D), lambda i: (0, i, 0))
_w_spec = pl.BlockSpec((D, D), lambda i: (0, 0))
_b_spec = pl.BlockSpec((1, D), lambda i: (0, 0))
_out_shape = jax.ShapeDtypeStruct((N, D), jnp.float32)

_mm_call = pl.pallas_call(
    _mm_tc, grid=_GRID,
    in_specs=[_row_spec, _w_spec],
    out_specs=_row_spec, out_shape=_out_shape)

_scale_call = pl.pallas_call(
    _scale_tc, grid=_GRID,
    in_specs=[_deg_spec, _row_spec],
    out_specs=_row_spec, out_shape=_out_shape)

_mid_call = pl.pallas_call(
    _mid_tc, grid=_GRID,
    in_specs=[_deg_spec, _s_spec, _row_spec, _b_spec, _w_spec],
    out_specs=_row_spec, out_shape=_out_shape)

_last_call = pl.pallas_call(
    _last_tc, grid=_GRID,
    in_specs=[_deg_spec, _s_spec, _row_spec, _b_spec, _w_spec, _b_spec],
    out_specs=_row_spec, out_shape=_out_shape)


def kernel(x, edge_index, W1, b1, W2, b2, Wfc, bfc):
    ei = edge_index.astype(jnp.int32)
    pad = EPAD - E
    r = jnp.arange(pad, dtype=jnp.int32)
    src = jnp.concatenate([ei[0], r * 41 % N])
    dst = jnp.concatenate([ei[1], N + r % (NPAD - N)])
    hi = jnp.right_shift(dst, 3)
    lo = jnp.bitwise_and(dst, TREP * 8 - 1)
    onehot = jnp.tile(jnp.repeat(jnp.eye(8, dtype=jnp.float32), 16, axis=1),
                      (TREP, 1))
    p1 = _mm_call(x, W1)
    degp = _deg_sc(onehot, lo, hi)
    deg = degp.reshape(NC, PK, 8, 16)[:, :, :, 0].reshape(NC, NPAD)[:, :N]
    degT = jnp.transpose(deg)
    g1 = _scale_call(degT, p1)
    s1 = _agg_sc(g1, src, dst)
    g2 = _mid_call(degT, s1, g1, b1.reshape(1, D), W2)
    s2 = _agg_sc(g2, src, dst)
    out = _last_call(degT, s2, g2, b2.reshape(1, D), Wfc, bfc.reshape(1, D))
    return out
